# Initial kernel scaffold; baseline (speedup 1.0000x reference)
import functools

import jax
import jax.numpy as jnp
from jax import lax
from jax.experimental import pallas as pl
from jax.experimental.pallas import tpu as pltpu
from jax.experimental.pallas import tpu_sc as plsc

CHUNK = 125
NUM_CORES = 2
NUM_SUBCORES = 16
NW = NUM_CORES * NUM_SUBCORES


def _sc_degree(dst2d, zeros_np, n_pad):
    nch = dst2d.shape[0]
    cpw = nch // NW
    rps = n_pad // NUM_SUBCORES
    mesh = plsc.VectorSubcoreMesh(core_axis_name="c", subcore_axis_name="s")

    @functools.partial(
        pl.kernel,
        mesh=mesh,
        out_type=jax.ShapeDtypeStruct((NUM_CORES, n_pad), jnp.float32),
        scratch_types=[
            pltpu.VMEM((cpw, CHUNK), jnp.int32),
            pltpu.VMEM((CHUNK,), jnp.float32),
            pltpu.VMEM_SHARED((n_pad,), jnp.float32),
        ],
    )
    def deg_kernel(dst_hbm, z_hbm, out_hbm, dst_v, ones_v, acc):
        c = lax.axis_index("c")
        s = lax.axis_index("s")
        wid = s * NUM_CORES + c
        pltpu.sync_copy(z_hbm.at[pl.ds(s * rps, rps)], acc.at[pl.ds(s * rps, rps)])
        pltpu.sync_copy(dst_hbm.at[pl.ds(wid * cpw, cpw)], dst_v)
        for i in range(CHUNK // 16):
            ones_v[pl.ds(i * 16, 16)] = jnp.full((16,), 1.0, jnp.float32)
        ones_v[pl.ds(CHUNK - 16, 16)] = jnp.full((16,), 1.0, jnp.float32)
        plsc.subcore_barrier()

        def body(j, carry):
            pltpu.sync_copy(ones_v, acc.at[dst_v.at[j]], add=True)
            return carry

        lax.fori_loop(0, cpw, body, 0)
        plsc.subcore_barrier()
        pltpu.sync_copy(acc.at[pl.ds(s * rps, rps)],
                        out_hbm.at[c, pl.ds(s * rps, rps)])

    return deg_kernel(dst2d, zeros_np)


def _sc_aggregate(hs, src2d, dst2d, zeros_nf):
    n, f = hs.shape
    nch = src2d.shape[0]
    cpw = nch // NW
    rps = n // NUM_SUBCORES
    mesh = plsc.VectorSubcoreMesh(core_axis_name="c", subcore_axis_name="s")

    @functools.partial(
        pl.kernel,
        mesh=mesh,
        out_type=jax.ShapeDtypeStruct((NUM_CORES, n, f), jnp.float32),
        scratch_types=[
            pltpu.VMEM((cpw, CHUNK), jnp.int32),
            pltpu.VMEM((cpw, CHUNK), jnp.int32),
            pltpu.VMEM((CHUNK, f), jnp.float32),
            pltpu.VMEM_SHARED((n, f), jnp.float32),
            pltpu.SemaphoreType.DMA,
        ],
    )
    def agg_kernel(hs_hbm, src_hbm, dst_hbm, z_hbm, out_hbm,
                   src_v, dst_v, rows_v, acc, sem):
        c = lax.axis_index("c")
        s = lax.axis_index("s")
        wid = s * NUM_CORES + c
        pltpu.sync_copy(z_hbm.at[pl.ds(s * rps, rps)], acc.at[pl.ds(s * rps, rps)])
        pltpu.sync_copy(src_hbm.at[pl.ds(wid * cpw, cpw)], src_v)
        pltpu.sync_copy(dst_hbm.at[pl.ds(wid * cpw, cpw)], dst_v)
        plsc.subcore_barrier()

        def body(j, carry):
            pltpu.async_copy(hs_hbm.at[src_v.at[j]], rows_v, sem).wait()
            pltpu.sync_copy(rows_v, acc.at[dst_v.at[j]], add=True)
            return carry

        lax.fori_loop(0, cpw, body, 0)
        plsc.subcore_barrier()
        pltpu.sync_copy(acc.at[pl.ds(s * rps, rps)],
                        out_hbm.at[c, pl.ds(s * rps, rps)])

    return agg_kernel(hs, src2d, dst2d, zeros_nf)


def _tc_layer1(x, w1, d0, d1, block_n=2000):
    n, f_in = x.shape
    hid = w1.shape[1]
    grid = n // block_n

    def body(x_ref, w_ref, d0_ref, d1_ref, hs_ref, dinv_ref):
        deg = d0_ref[...] + d1_ref[...] + 1.0
        dinv = lax.rsqrt(deg)
        hw = jnp.dot(x_ref[...], w_ref[...], preferred_element_type=jnp.float32)
        hs_ref[...] = hw * dinv
        dinv_ref[...] = dinv

    return pl.pallas_call(
        body,
        grid=(grid,),
        in_specs=[
            pl.BlockSpec((block_n, f_in), lambda i: (i, 0)),
            pl.BlockSpec((f_in, hid), lambda i: (0, 0)),
            pl.BlockSpec((block_n, 1), lambda i: (i, 0)),
            pl.BlockSpec((block_n, 1), lambda i: (i, 0)),
        ],
        out_specs=[
            pl.BlockSpec((block_n, hid), lambda i: (i, 0)),
            pl.BlockSpec((block_n, 1), lambda i: (i, 0)),
        ],
        out_shape=[
            jax.ShapeDtypeStruct((n, hid), jnp.float32),
            jax.ShapeDtypeStruct((n, 1), jnp.float32),
        ],
    )(x, w1, d0, d1)


def _tc_layer2(a0, a1, hs1, dinv, b1, w2p, block_n=2000):
    n, hid = hs1.shape
    cp = w2p.shape[1]
    grid = n // block_n

    def body(a0_ref, a1_ref, hs_ref, dinv_ref, b1_ref, w2_ref, out_ref):
        dinv = dinv_ref[...]
        t = dinv * (a0_ref[...] + a1_ref[...] + hs_ref[...]) + b1_ref[...]
        h = jnp.maximum(t, 0.0)
        hw2 = jnp.dot(h, w2_ref[...], preferred_element_type=jnp.float32)
        out_ref[...] = hw2 * dinv

    return pl.pallas_call(
        body,
        grid=(grid,),
        in_specs=[
            pl.BlockSpec((block_n, hid), lambda i: (i, 0)),
            pl.BlockSpec((block_n, hid), lambda i: (i, 0)),
            pl.BlockSpec((block_n, hid), lambda i: (i, 0)),
            pl.BlockSpec((block_n, 1), lambda i: (i, 0)),
            pl.BlockSpec((1, hid), lambda i: (0, 0)),
            pl.BlockSpec((hid, cp), lambda i: (0, 0)),
        ],
        out_specs=pl.BlockSpec((block_n, cp), lambda i: (i, 0)),
        out_shape=jax.ShapeDtypeStruct((n, cp), jnp.float32),
    )(a0, a1, hs1, dinv, b1, w2p)


def _tc_final(q0, q1, hs2, dinv, b2p, c_out, block_n=2000):
    n, cp = hs2.shape
    grid = n // block_n

    def body(q0_ref, q1_ref, hs_ref, dinv_ref, b2_ref, out_ref):
        t = dinv_ref[...] * (q0_ref[...] + q1_ref[...] + hs_ref[...]) + b2_ref[...]
        col = lax.broadcasted_iota(jnp.int32, t.shape, 1)
        tm = jnp.where(col < c_out, t, -jnp.inf)
        m = jnp.max(tm, axis=1, keepdims=True)
        lse = jnp.log(jnp.sum(jnp.exp(tm - m), axis=1, keepdims=True)) + m
        out_ref[...] = (tm - lse)[:, :c_out]

    return pl.pallas_call(
        body,
        grid=(grid,),
        in_specs=[
            pl.BlockSpec((block_n, cp), lambda i: (i, 0)),
            pl.BlockSpec((block_n, cp), lambda i: (i, 0)),
            pl.BlockSpec((block_n, cp), lambda i: (i, 0)),
            pl.BlockSpec((block_n, 1), lambda i: (i, 0)),
            pl.BlockSpec((1, cp), lambda i: (0, 0)),
        ],
        out_specs=pl.BlockSpec((block_n, c_out), lambda i: (i, 0)),
        out_shape=jax.ShapeDtypeStruct((n, c_out), jnp.float32),
    )(q0, q1, hs2, dinv, b2p)


def kernel(x, edge_index, W1, b1, W2, b2):
    n, f_in = x.shape
    e = edge_index.shape[1]
    hid = W1.shape[1]
    c_out = W2.shape[1]
    cp = 48
    n_pad = 10240

    src2d = edge_index[0].reshape(e // CHUNK, CHUNK)
    dst2d = edge_index[1].reshape(e // CHUNK, CHUNK)

    zeros_np = jnp.zeros((n_pad,), jnp.float32)
    zeros_nh = jnp.zeros((n, hid), jnp.float32)
    zeros_nc = jnp.zeros((n, cp), jnp.float32)
    w2p = jnp.pad(W2, ((0, 0), (0, cp - c_out)))
    b1r = b1.reshape(1, hid)
    b2p = jnp.pad(b2, (0, cp - c_out)).reshape(1, cp)

    degp = _sc_degree(dst2d, zeros_np, n_pad)
    d0 = degp[0, :n].reshape(n, 1)
    d1 = degp[1, :n].reshape(n, 1)

    hs1, dinv = _tc_layer1(x, W1, d0, d1)
    p = _sc_aggregate(hs1, src2d, dst2d, zeros_nh)
    hs2 = _tc_layer2(p[0], p[1], hs1, dinv, b1r, w2p)
    q = _sc_aggregate(hs2, src2d, dst2d, zeros_nc)
    return _tc_final(q[0], q[1], hs2, dinv, b2p, c_out)


# speedup vs baseline: 14.4743x; 14.4743x over previous
import functools

import jax
import jax.numpy as jnp
from jax import lax
from jax.experimental import pallas as pl
from jax.experimental.pallas import tpu as pltpu
from jax.experimental.pallas import tpu_sc as plsc

CHUNK = 128
NUM_CORES = 2
NUM_SUBCORES = 16
NW = NUM_CORES * NUM_SUBCORES
N_PAD = 10240
N_PAD_DEG = 16384


def _sc_degree(dst2d, zeros_np):
    nch = dst2d.shape[0]
    cpw = nch // NW
    rps = N_PAD_DEG // NUM_SUBCORES
    mesh = plsc.VectorSubcoreMesh(core_axis_name="c", subcore_axis_name="s")

    @functools.partial(
        pl.kernel,
        mesh=mesh,
        out_type=jax.ShapeDtypeStruct((NUM_CORES * N_PAD_DEG,), jnp.float32),
        compiler_params=pltpu.CompilerParams(use_tc_tiling_on_sc=False),
        scratch_types=[
            pltpu.VMEM((cpw, CHUNK), jnp.int32),
            pltpu.VMEM((CHUNK,), jnp.float32),
            pltpu.VMEM_SHARED((N_PAD_DEG,), jnp.float32),
        ],
    )
    def deg_kernel(dst_hbm, z_hbm, out_hbm, dst_v, ones_v, acc):
        c = lax.axis_index("c")
        s = lax.axis_index("s")
        wid = s * NUM_CORES + c
        pltpu.sync_copy(z_hbm.at[pl.ds(s * rps, rps)], acc.at[pl.ds(s * rps, rps)])
        pltpu.sync_copy(dst_hbm.at[pl.ds(wid * cpw, cpw)], dst_v)
        for i in range(CHUNK // 16):
            ones_v[pl.ds(i * 16, 16)] = jnp.full((16,), 1.0, jnp.float32)
        plsc.subcore_barrier()

        def body(j, carry):
            pltpu.sync_copy(ones_v, acc.at[dst_v.at[j]], add=True)
            return carry

        lax.fori_loop(0, cpw, body, 0)
        plsc.subcore_barrier()
        pltpu.sync_copy(acc.at[pl.ds(s * rps, rps)],
                        out_hbm.at[pl.ds(c * N_PAD_DEG + s * rps, rps)])

    return deg_kernel(dst2d, zeros_np)


def _sc_aggregate(hs, src2d, dst2d, zeros_nf):
    n, f = hs.shape
    nch = src2d.shape[0]
    cpw = nch // NW
    rps = N_PAD // NUM_SUBCORES
    mesh = plsc.VectorSubcoreMesh(core_axis_name="c", subcore_axis_name="s")

    @functools.partial(
        pl.kernel,
        mesh=mesh,
        out_type=jax.ShapeDtypeStruct((NUM_CORES, N_PAD, f), jnp.float32),
        compiler_params=pltpu.CompilerParams(use_tc_tiling_on_sc=False),
        scratch_types=[
            pltpu.VMEM((cpw, CHUNK), jnp.int32),
            pltpu.VMEM((cpw, CHUNK), jnp.int32),
            pltpu.VMEM((CHUNK, f), jnp.float32),
            pltpu.VMEM_SHARED((N_PAD, f), jnp.float32),
            pltpu.SemaphoreType.DMA,
        ],
    )
    def agg_kernel(hs_hbm, src_hbm, dst_hbm, z_hbm, out_hbm,
                   src_v, dst_v, rows_v, acc, sem):
        c = lax.axis_index("c")
        s = lax.axis_index("s")
        wid = s * NUM_CORES + c
        pltpu.sync_copy(z_hbm.at[pl.ds(s * rps, rps)], acc.at[pl.ds(s * rps, rps)])
        pltpu.sync_copy(src_hbm.at[pl.ds(wid * cpw, cpw)], src_v)
        pltpu.sync_copy(dst_hbm.at[pl.ds(wid * cpw, cpw)], dst_v)
        plsc.subcore_barrier()

        def body(j, carry):
            pltpu.async_copy(hs_hbm.at[src_v.at[j]], rows_v, sem).wait()
            pltpu.sync_copy(rows_v, acc.at[dst_v.at[j]], add=True)
            return carry

        lax.fori_loop(0, cpw, body, 0)
        plsc.subcore_barrier()
        pltpu.sync_copy(acc.at[pl.ds(s * rps, rps), :],
                        out_hbm.at[c, pl.ds(s * rps, rps), :])

    return agg_kernel(hs, src2d, dst2d, zeros_nf)


def _tc_layer1(x, w1, d0, d1, block_n=2000):
    n, f_in = x.shape
    hid = w1.shape[1]
    grid = n // block_n

    def body(x_ref, w_ref, d0_ref, d1_ref, hs_ref, dinv_ref):
        deg = d0_ref[...] + d1_ref[...] + 1.0
        dinv = lax.rsqrt(deg)
        hw = jnp.dot(x_ref[...], w_ref[...], preferred_element_type=jnp.float32)
        hs_ref[...] = hw * dinv
        dinv_ref[...] = dinv

    return pl.pallas_call(
        body,
        grid=(grid,),
        in_specs=[
            pl.BlockSpec((block_n, f_in), lambda i: (i, 0)),
            pl.BlockSpec((f_in, hid), lambda i: (0, 0)),
            pl.BlockSpec((block_n, 1), lambda i: (i, 0)),
            pl.BlockSpec((block_n, 1), lambda i: (i, 0)),
        ],
        out_specs=[
            pl.BlockSpec((block_n, hid), lambda i: (i, 0)),
            pl.BlockSpec((block_n, 1), lambda i: (i, 0)),
        ],
        out_shape=[
            jax.ShapeDtypeStruct((n, hid), jnp.float32),
            jax.ShapeDtypeStruct((n, 1), jnp.float32),
        ],
    )(x, w1, d0, d1)


def _tc_layer2(a0, a1, hs1, dinv, b1, w2p, block_n=2000):
    n, hid = hs1.shape
    cp = w2p.shape[1]
    grid = n // block_n

    def body(a0_ref, a1_ref, hs_ref, dinv_ref, b1_ref, w2_ref, out_ref):
        dinv = dinv_ref[...]
        t = dinv * (a0_ref[...] + a1_ref[...] + hs_ref[...]) + b1_ref[...]
        h = jnp.maximum(t, 0.0)
        hw2 = jnp.dot(h, w2_ref[...], preferred_element_type=jnp.float32)
        out_ref[...] = hw2 * dinv

    return pl.pallas_call(
        body,
        grid=(grid,),
        in_specs=[
            pl.BlockSpec((block_n, hid), lambda i: (i, 0)),
            pl.BlockSpec((block_n, hid), lambda i: (i, 0)),
            pl.BlockSpec((block_n, hid), lambda i: (i, 0)),
            pl.BlockSpec((block_n, 1), lambda i: (i, 0)),
            pl.BlockSpec((1, hid), lambda i: (0, 0)),
            pl.BlockSpec((hid, cp), lambda i: (0, 0)),
        ],
        out_specs=pl.BlockSpec((block_n, cp), lambda i: (i, 0)),
        out_shape=jax.ShapeDtypeStruct((n, cp), jnp.float32),
    )(a0, a1, hs1, dinv, b1, w2p)


def _tc_final(q0, q1, hs2, dinv, b2p, c_out, block_n=2000):
    n, cp = hs2.shape
    grid = n // block_n

    def body(q0_ref, q1_ref, hs_ref, dinv_ref, b2_ref, out_ref):
        t = dinv_ref[...] * (q0_ref[...] + q1_ref[...] + hs_ref[...]) + b2_ref[...]
        col = lax.broadcasted_iota(jnp.int32, t.shape, 1)
        tm = jnp.where(col < c_out, t, -jnp.inf)
        m = jnp.max(tm, axis=1, keepdims=True)
        lse = jnp.log(jnp.sum(jnp.exp(tm - m), axis=1, keepdims=True)) + m
        out_ref[...] = (tm - lse)[:, :c_out]

    return pl.pallas_call(
        body,
        grid=(grid,),
        in_specs=[
            pl.BlockSpec((block_n, cp), lambda i: (i, 0)),
            pl.BlockSpec((block_n, cp), lambda i: (i, 0)),
            pl.BlockSpec((block_n, cp), lambda i: (i, 0)),
            pl.BlockSpec((block_n, 1), lambda i: (i, 0)),
            pl.BlockSpec((1, cp), lambda i: (0, 0)),
        ],
        out_specs=pl.BlockSpec((block_n, c_out), lambda i: (i, 0)),
        out_shape=jax.ShapeDtypeStruct((n, c_out), jnp.float32),
    )(q0, q1, hs2, dinv, b2p)


def kernel(x, edge_index, W1, b1, W2, b2):
    n, f_in = x.shape
    e = edge_index.shape[1]
    hid = W1.shape[1]
    c_out = W2.shape[1]
    cp = 48

    e_pad = ((e + CHUNK * NW - 1) // (CHUNK * NW)) * (CHUNK * NW)
    pad = e_pad - e
    src_full = jnp.concatenate([edge_index[0], jnp.zeros((pad,), jnp.int32)])
    dst_full = jnp.concatenate(
        [edge_index[1], jnp.full((pad,), n, jnp.int32)])
    src2d = src_full.reshape(e_pad // CHUNK, CHUNK)
    dst2d = dst_full.reshape(e_pad // CHUNK, CHUNK)

    zeros_np = jnp.zeros((N_PAD_DEG,), jnp.float32)
    zeros_nh = jnp.zeros((N_PAD, hid), jnp.float32)
    zeros_nc = jnp.zeros((N_PAD, cp), jnp.float32)
    w2p = jnp.pad(W2, ((0, 0), (0, cp - c_out)))
    b1r = b1.reshape(1, hid)
    b2p = jnp.pad(b2, (0, cp - c_out)).reshape(1, cp)

    degp = _sc_degree(dst2d, zeros_np)
    d0 = degp[:n].reshape(n, 1)
    d1 = degp[N_PAD_DEG:N_PAD_DEG + n].reshape(n, 1)

    hs1, dinv = _tc_layer1(x, W1, d0, d1)
    p = _sc_aggregate(hs1, src2d, dst2d, zeros_nh)
    hs2 = _tc_layer2(p[0, :n], p[1, :n], hs1, dinv, b1r, w2p)
    q = _sc_aggregate(hs2, src2d, dst2d, zeros_nc)
    return _tc_final(q[0, :n], q[1, :n], hs2, dinv, b2p, c_out)


# speedup vs baseline: 24.3558x; 1.6827x over previous
import functools

import jax
import jax.numpy as jnp
from jax import lax
from jax.experimental import pallas as pl
from jax.experimental.pallas import tpu as pltpu
from jax.experimental.pallas import tpu_sc as plsc

CHUNK = 128
NUM_CORES = 2
NUM_SUBCORES = 16
NW = NUM_CORES * NUM_SUBCORES
N_PAD = 10240
N_PAD_DEG = 16384


def _sc_degree(dst2d, zeros_np):
    nch = dst2d.shape[0]
    cpw = nch // NW
    rps = N_PAD_DEG // NUM_SUBCORES
    mesh = plsc.VectorSubcoreMesh(core_axis_name="c", subcore_axis_name="s")

    @functools.partial(
        pl.kernel,
        mesh=mesh,
        out_type=jax.ShapeDtypeStruct((NUM_CORES * N_PAD_DEG,), jnp.float32),
        compiler_params=pltpu.CompilerParams(use_tc_tiling_on_sc=False),
        scratch_types=[
            pltpu.VMEM((cpw, CHUNK), jnp.int32),
            pltpu.VMEM((CHUNK,), jnp.float32),
            pltpu.VMEM_SHARED((N_PAD_DEG,), jnp.float32),
            pltpu.SemaphoreType.DMA,
        ],
    )
    def deg_kernel(dst_hbm, z_hbm, out_hbm, dst_v, ones_v, acc, sem):
        c = lax.axis_index("c")
        s = lax.axis_index("s")
        wid = s * NUM_CORES + c
        pltpu.sync_copy(z_hbm.at[pl.ds(s * rps, rps)], acc.at[pl.ds(s * rps, rps)])
        pltpu.sync_copy(dst_hbm.at[pl.ds(wid * cpw, cpw)], dst_v)
        for i in range(CHUNK // 16):
            ones_v[pl.ds(i * 16, 16)] = jnp.full((16,), 1.0, jnp.float32)
        plsc.subcore_barrier()

        def fire(j, carry):
            pltpu.make_async_copy(ones_v, acc.at[dst_v.at[j]], sem).start(add=True)
            return carry

        lax.fori_loop(0, cpw, fire, 0)

        def drain(j, carry):
            pltpu.make_async_copy(ones_v, acc.at[pl.ds(0, CHUNK)], sem).wait()
            return carry

        lax.fori_loop(0, cpw, drain, 0)
        plsc.subcore_barrier()
        pltpu.sync_copy(acc.at[pl.ds(s * rps, rps)],
                        out_hbm.at[pl.ds(c * N_PAD_DEG + s * rps, rps)])

    return deg_kernel(dst2d, zeros_np)


def _sc_aggregate(hs, src2d, dst2d, zeros_nf):
    n, f = hs.shape
    nch = src2d.shape[0]
    cpw = nch // NW
    epw = cpw * CHUNK
    rps = N_PAD // NUM_SUBCORES
    mesh = plsc.VectorSubcoreMesh(core_axis_name="c", subcore_axis_name="s")

    @functools.partial(
        pl.kernel,
        mesh=mesh,
        out_type=jax.ShapeDtypeStruct((NUM_CORES, N_PAD, f), jnp.float32),
        compiler_params=pltpu.CompilerParams(use_tc_tiling_on_sc=False),
        scratch_types=[
            pltpu.VMEM((cpw, CHUNK), jnp.int32),
            pltpu.VMEM((cpw, CHUNK), jnp.int32),
            pltpu.VMEM((cpw * CHUNK, f), jnp.float32),
            pltpu.VMEM_SHARED((N_PAD, f), jnp.float32),
            pltpu.SemaphoreType.DMA,
            pltpu.SemaphoreType.DMA,
        ],
    )
    def agg_kernel(hs_hbm, src_hbm, dst_hbm, z_hbm, out_hbm,
                   src_v, dst_v, rows_v, acc, gsem, ssem):
        c = lax.axis_index("c")
        s = lax.axis_index("s")
        wid = s * NUM_CORES + c
        pltpu.sync_copy(z_hbm.at[pl.ds(s * rps, rps)], acc.at[pl.ds(s * rps, rps)])
        pltpu.sync_copy(src_hbm.at[pl.ds(wid * cpw, cpw)], src_v)
        pltpu.sync_copy(dst_hbm.at[pl.ds(wid * cpw, cpw)], dst_v)
        plsc.subcore_barrier()

        def fire_gather(j, carry):
            pltpu.make_async_copy(
                hs_hbm.at[src_v.at[j]],
                rows_v.at[pl.ds(j * CHUNK, CHUNK), :], gsem).start()
            return carry

        lax.fori_loop(0, cpw, fire_gather, 0)
        pltpu.make_async_copy(hs_hbm.at[pl.ds(0, epw), :], rows_v, gsem).wait()

        def fire_scatter(j, carry):
            pltpu.make_async_copy(
                rows_v.at[pl.ds(j * CHUNK, CHUNK), :],
                acc.at[dst_v.at[j]], ssem).start(add=True)
            return carry

        lax.fori_loop(0, cpw, fire_scatter, 0)
        pltpu.make_async_copy(rows_v, acc.at[pl.ds(0, epw), :], ssem).wait()
        plsc.subcore_barrier()
        pltpu.sync_copy(acc.at[pl.ds(s * rps, rps), :],
                        out_hbm.at[c, pl.ds(s * rps, rps), :])

    return agg_kernel(hs, src2d, dst2d, zeros_nf)


def _tc_layer1(x, w1, d0, d1, block_n=2000):
    n, f_in = x.shape
    hid = w1.shape[1]
    grid = n // block_n

    def body(x_ref, w_ref, d0_ref, d1_ref, hs_ref, dinv_ref):
        deg = d0_ref[...] + d1_ref[...] + 1.0
        dinv = lax.rsqrt(deg)
        hw = jnp.dot(x_ref[...], w_ref[...], preferred_element_type=jnp.float32)
        hs_ref[...] = hw * dinv
        dinv_ref[...] = dinv

    return pl.pallas_call(
        body,
        grid=(grid,),
        in_specs=[
            pl.BlockSpec((block_n, f_in), lambda i: (i, 0)),
            pl.BlockSpec((f_in, hid), lambda i: (0, 0)),
            pl.BlockSpec((block_n, 1), lambda i: (i, 0)),
            pl.BlockSpec((block_n, 1), lambda i: (i, 0)),
        ],
        out_specs=[
            pl.BlockSpec((block_n, hid), lambda i: (i, 0)),
            pl.BlockSpec((block_n, 1), lambda i: (i, 0)),
        ],
        out_shape=[
            jax.ShapeDtypeStruct((n, hid), jnp.float32),
            jax.ShapeDtypeStruct((n, 1), jnp.float32),
        ],
    )(x, w1, d0, d1)


def _tc_layer2(a0, a1, hs1, dinv, b1, block_n=2000):
    n, hid = hs1.shape
    grid = n // block_n

    def body(a0_ref, a1_ref, hs_ref, dinv_ref, b1_ref, out_ref):
        dinv = dinv_ref[...]
        t = dinv * (a0_ref[...] + a1_ref[...] + hs_ref[...]) + b1_ref[...]
        out_ref[...] = jnp.maximum(t, 0.0) * dinv

    return pl.pallas_call(
        body,
        grid=(grid,),
        in_specs=[
            pl.BlockSpec((block_n, hid), lambda i: (i, 0)),
            pl.BlockSpec((block_n, hid), lambda i: (i, 0)),
            pl.BlockSpec((block_n, hid), lambda i: (i, 0)),
            pl.BlockSpec((block_n, 1), lambda i: (i, 0)),
            pl.BlockSpec((1, hid), lambda i: (0, 0)),
        ],
        out_specs=pl.BlockSpec((block_n, hid), lambda i: (i, 0)),
        out_shape=jax.ShapeDtypeStruct((n, hid), jnp.float32),
    )(a0, a1, hs1, dinv, b1)


def _tc_final(q0, q1, hs2, dinv, w2, b2, block_n=2000):
    n, hid = hs2.shape
    c_out = w2.shape[1]
    grid = n // block_n

    def body(q0_ref, q1_ref, hs_ref, dinv_ref, w2_ref, b2_ref, out_ref):
        t = dinv_ref[...] * (q0_ref[...] + q1_ref[...] + hs_ref[...])
        logits = jnp.dot(t, w2_ref[...],
                         preferred_element_type=jnp.float32) + b2_ref[...]
        m = jnp.max(logits, axis=1, keepdims=True)
        lse = jnp.log(jnp.sum(jnp.exp(logits - m), axis=1, keepdims=True)) + m
        out_ref[...] = logits - lse

    return pl.pallas_call(
        body,
        grid=(grid,),
        in_specs=[
            pl.BlockSpec((block_n, hid), lambda i: (i, 0)),
            pl.BlockSpec((block_n, hid), lambda i: (i, 0)),
            pl.BlockSpec((block_n, hid), lambda i: (i, 0)),
            pl.BlockSpec((block_n, 1), lambda i: (i, 0)),
            pl.BlockSpec((hid, c_out), lambda i: (0, 0)),
            pl.BlockSpec((1, c_out), lambda i: (0, 0)),
        ],
        out_specs=pl.BlockSpec((block_n, c_out), lambda i: (i, 0)),
        out_shape=jax.ShapeDtypeStruct((n, c_out), jnp.float32),
    )(q0, q1, hs2, dinv, w2, b2)


def kernel(x, edge_index, W1, b1, W2, b2):
    n, f_in = x.shape
    e = edge_index.shape[1]
    hid = W1.shape[1]
    c_out = W2.shape[1]

    e_pad = ((e + CHUNK * NW - 1) // (CHUNK * NW)) * (CHUNK * NW)
    pad = e_pad - e
    src_full = jnp.concatenate([edge_index[0], jnp.zeros((pad,), jnp.int32)])
    dst_full = jnp.concatenate(
        [edge_index[1], jnp.full((pad,), n, jnp.int32)])
    src2d = src_full.reshape(e_pad // CHUNK, CHUNK)
    dst2d = dst_full.reshape(e_pad // CHUNK, CHUNK)

    zeros_np = jnp.zeros((N_PAD_DEG,), jnp.float32)
    zeros_nh = jnp.zeros((N_PAD, hid), jnp.float32)
    b1r = b1.reshape(1, hid)
    b2r = b2.reshape(1, c_out)

    degp = _sc_degree(dst2d, zeros_np)
    d0 = degp[:n].reshape(n, 1)
    d1 = degp[N_PAD_DEG:N_PAD_DEG + n].reshape(n, 1)

    hs1, dinv = _tc_layer1(x, W1, d0, d1)
    p = _sc_aggregate(hs1, src2d, dst2d, zeros_nh)
    hs2 = _tc_layer2(p[0, :n], p[1, :n], hs1, dinv, b1r)
    q = _sc_aggregate(hs2, src2d, dst2d, zeros_nh)
    return _tc_final(q[0, :n], q[1, :n], hs2, dinv, W2, b2r)


# speedup vs baseline: 25.6234x; 1.0520x over previous
import functools

import jax
import jax.numpy as jnp
from jax import lax
from jax.experimental import pallas as pl
from jax.experimental.pallas import tpu as pltpu
from jax.experimental.pallas import tpu_sc as plsc

CHUNK = 128
NUM_CORES = 2
NUM_SUBCORES = 16
NW = NUM_CORES * NUM_SUBCORES
N_PAD = 10240
N_PAD_DEG = 16384
RPS = N_PAD // NUM_SUBCORES
_SC_PARAMS = pltpu.CompilerParams(use_tc_tiling_on_sc=False,
                                  needs_layout_passes=False)


def _rsqrt16(x):
    i = plsc.bitcast(x, jnp.int32)
    i = jnp.int32(0x5F3759DF) - (i >> 1)
    y = plsc.bitcast(i, jnp.float32)
    for _ in range(3):
        y = y * (1.5 - 0.5 * x * y * y)
    return y


def _fill_ones(ones_v):
    for i in range(CHUNK // 16):
        ones_v[pl.ds(i * 16, 16)] = jnp.full((16,), 1.0, jnp.float32)


WAVE = 8


def _run_aggregation(table, src_v, dst_v, rows_v, acc, gsems, ssems, cpw):
    nwaves = cpw // WAVE
    wrows = WAVE * CHUNK

    def fire_gathers(w, b):
        def fire(j, carry):
            pltpu.make_async_copy(
                table.at[src_v.at[w * WAVE + j]],
                rows_v.at[b, pl.ds(j * CHUNK, CHUNK), :], gsems[b]).start()
            return carry
        lax.fori_loop(0, WAVE, fire, 0)

    def fire_scatters(w, b):
        def fire(j, carry):
            pltpu.make_async_copy(
                rows_v.at[b, pl.ds(j * CHUNK, CHUNK), :],
                acc.at[dst_v.at[w * WAVE + j]], ssems[b]).start(add=True)
            return carry
        lax.fori_loop(0, WAVE, fire, 0)

    def drain(sem, buf_side):
        def w1(j, carry):
            pltpu.make_async_copy(
                table.at[pl.ds(0, CHUNK), :],
                rows_v.at[buf_side, pl.ds(0, CHUNK), :], sem).wait()
            return carry
        lax.fori_loop(0, WAVE, w1, 0)

    fire_gathers(0, 0)
    for w in range(nwaves):
        b = w & 1
        if w + 1 < nwaves:
            if w >= 1:
                drain(ssems[1 - b], 1 - b)
            fire_gathers(w + 1, 1 - b)
        drain(gsems[b], b)
        fire_scatters(w, b)
    drain(ssems[(nwaves - 1) & 1], (nwaves - 1) & 1)
    if nwaves >= 2:
        drain(ssems[(nwaves - 2) & 1], (nwaves - 2) & 1)


def _sc_mega1(hw1, src2d, dst2d, z16, zdeg):
    nch = src2d.shape[0]
    cpw = nch // NW
    dpw = nch // NUM_SUBCORES
    hid = hw1.shape[1]
    drps = N_PAD_DEG // NUM_SUBCORES
    mesh = plsc.VectorSubcoreMesh(core_axis_name="c", subcore_axis_name="s")

    @functools.partial(
        pl.kernel,
        mesh=mesh,
        out_type=(
            jax.ShapeDtypeStruct((NUM_CORES, N_PAD, hid), jnp.float32),
            jax.ShapeDtypeStruct((NUM_CORES, N_PAD, hid), jnp.float32),
            jax.ShapeDtypeStruct((N_PAD,), jnp.float32),
        ),
        compiler_params=_SC_PARAMS,
        scratch_types=[
            pltpu.VMEM((dpw, CHUNK), jnp.int32),
            pltpu.VMEM((cpw, CHUNK), jnp.int32),
            pltpu.VMEM((cpw, CHUNK), jnp.int32),
            pltpu.VMEM((CHUNK,), jnp.float32),
            pltpu.VMEM((RPS, 16), jnp.float32),
            pltpu.VMEM((RPS,), jnp.float32),
            pltpu.VMEM((RPS,), jnp.float32),
            pltpu.VMEM((2, WAVE * CHUNK, 16), jnp.float32),
            pltpu.VMEM_SHARED((N_PAD, 16), jnp.float32),
            pltpu.VMEM_SHARED((N_PAD_DEG,), jnp.float32),
            pltpu.SemaphoreType.DMA,
            pltpu.SemaphoreType.DMA,
            pltpu.SemaphoreType.DMA,
            pltpu.SemaphoreType.DMA,
            pltpu.SemaphoreType.DMA,
        ],
    )
    def mega1(hw_hbm, src_hbm, dst_hbm, z16_hbm, zd_hbm,
              p_hbm, hs1_hbm, dinv_hbm,
              degidx_v, src_v, dst_v, ones_v, hw_v, deg_v, dinv_v, rows_v,
              acc, dacc, dsem, gsem0, gsem1, ssem0, ssem1):
        c = lax.axis_index("c")
        s = lax.axis_index("s")
        wid = s * NUM_CORES + c
        pltpu.sync_copy(z16_hbm.at[pl.ds(s * RPS, RPS), :],
                        acc.at[pl.ds(s * RPS, RPS), :])
        pltpu.sync_copy(zd_hbm.at[pl.ds(s * drps, drps)],
                        dacc.at[pl.ds(s * drps, drps)])
        _fill_ones(ones_v)
        pltpu.sync_copy(dst_hbm.at[pl.ds(s * dpw, dpw)], degidx_v)
        plsc.subcore_barrier()

        def fire_deg(j, carry):
            pltpu.make_async_copy(ones_v, dacc.at[degidx_v.at[j]],
                                  dsem).start(add=True)
            return carry

        lax.fori_loop(0, dpw, fire_deg, 0)
        pltpu.sync_copy(hw_hbm.at[pl.ds(s * RPS, RPS), :], hw_v)
        pltpu.sync_copy(src_hbm.at[pl.ds(wid * cpw, cpw)], src_v)
        pltpu.sync_copy(dst_hbm.at[pl.ds(wid * cpw, cpw)], dst_v)

        def drain_deg(j, carry):
            pltpu.make_async_copy(ones_v, dacc.at[pl.ds(0, CHUNK)],
                                  dsem).wait()
            return carry

        lax.fori_loop(0, dpw, drain_deg, 0)
        plsc.subcore_barrier()

        pltpu.sync_copy(dacc.at[pl.ds(s * RPS, RPS)], deg_v)

        def dinv_blk(b, carry):
            dinv_v[pl.ds(b * 16, 16)] = _rsqrt16(
                deg_v[pl.ds(b * 16, 16)] + 1.0)
            return carry

        lax.fori_loop(0, RPS // 16, dinv_blk, 0)

        def scale_row(r, carry):
            bidx = jnp.zeros((16,), jnp.int32) + r
            db = plsc.load_gather(dinv_v, [bidx])
            hw_v[r, :] = hw_v[r, :] * db
            return carry

        lax.fori_loop(0, RPS, scale_row, 0)
        pltpu.sync_copy(hw_v, hs1_hbm.at[c, pl.ds(s * RPS, RPS), :])

        @pl.when(c == 0)
        def _():
            pltpu.sync_copy(dinv_v, dinv_hbm.at[pl.ds(s * RPS, RPS)])

        plsc.subcore_barrier()

        _run_aggregation(hs1_hbm.at[c], src_v, dst_v, rows_v, acc,
                         (gsem0, gsem1), (ssem0, ssem1), cpw)
        plsc.subcore_barrier()
        pltpu.sync_copy(acc.at[pl.ds(s * RPS, RPS), :],
                        p_hbm.at[c, pl.ds(s * RPS, RPS), :])

    return mega1(hw1, src2d, dst2d, z16, zdeg)


def _sc_mega2(p, hw1, dinv_flat, b1, src2d, dst2d, z16):
    nch = src2d.shape[0]
    cpw = nch // NW
    hid = hw1.shape[1]
    mesh = plsc.VectorSubcoreMesh(core_axis_name="c", subcore_axis_name="s")

    @functools.partial(
        pl.kernel,
        mesh=mesh,
        out_type=(
            jax.ShapeDtypeStruct((NUM_CORES, N_PAD, hid), jnp.float32),
            jax.ShapeDtypeStruct((NUM_CORES, N_PAD, hid), jnp.float32),
        ),
        compiler_params=_SC_PARAMS,
        scratch_types=[
            pltpu.VMEM((cpw, CHUNK), jnp.int32),
            pltpu.VMEM((cpw, CHUNK), jnp.int32),
            pltpu.VMEM((RPS, 16), jnp.float32),
            pltpu.VMEM((RPS, 16), jnp.float32),
            pltpu.VMEM((RPS, 16), jnp.float32),
            pltpu.VMEM((RPS,), jnp.float32),
            pltpu.VMEM((16,), jnp.float32),
            pltpu.VMEM((2, WAVE * CHUNK, 16), jnp.float32),
            pltpu.VMEM_SHARED((N_PAD, 16), jnp.float32),
            pltpu.SemaphoreType.DMA,
            pltpu.SemaphoreType.DMA,
            pltpu.SemaphoreType.DMA,
            pltpu.SemaphoreType.DMA,
        ],
    )
    def mega2(p_hbm, hw_hbm, dinv_hbm, b1_hbm, src_hbm, dst_hbm, z16_hbm,
              q_hbm, hs2_hbm,
              src_v, dst_v, p0_v, p1_v, hw_v, dinv_v, b1_v, rows_v,
              acc, gsem0, gsem1, ssem0, ssem1):
        c = lax.axis_index("c")
        s = lax.axis_index("s")
        wid = s * NUM_CORES + c
        pltpu.sync_copy(z16_hbm.at[pl.ds(s * RPS, RPS), :],
                        acc.at[pl.ds(s * RPS, RPS), :])
        pltpu.sync_copy(p_hbm.at[0, pl.ds(s * RPS, RPS), :], p0_v)
        pltpu.sync_copy(p_hbm.at[1, pl.ds(s * RPS, RPS), :], p1_v)
        pltpu.sync_copy(hw_hbm.at[pl.ds(s * RPS, RPS), :], hw_v)
        pltpu.sync_copy(dinv_hbm.at[pl.ds(s * RPS, RPS)], dinv_v)
        pltpu.sync_copy(b1_hbm, b1_v)
        pltpu.sync_copy(src_hbm.at[pl.ds(wid * cpw, cpw)], src_v)
        pltpu.sync_copy(dst_hbm.at[pl.ds(wid * cpw, cpw)], dst_v)
        b1v = b1_v[...]

        def row(r, carry):
            bidx = jnp.zeros((16,), jnp.int32) + r
            db = plsc.load_gather(dinv_v, [bidx])
            t = db * (p0_v[r, :] + p1_v[r, :] + db * hw_v[r, :]) + b1v
            hw_v[r, :] = jnp.maximum(t, 0.0) * db
            return carry

        lax.fori_loop(0, RPS, row, 0)
        pltpu.sync_copy(hw_v, hs2_hbm.at[c, pl.ds(s * RPS, RPS), :])
        plsc.subcore_barrier()

        _run_aggregation(hs2_hbm.at[c], src_v, dst_v, rows_v, acc,
                         (gsem0, gsem1), (ssem0, ssem1), cpw)
        plsc.subcore_barrier()
        pltpu.sync_copy(acc.at[pl.ds(s * RPS, RPS), :],
                        q_hbm.at[c, pl.ds(s * RPS, RPS), :])

    return mega2(p, hw1, dinv_flat, b1, src2d, dst2d, z16)


def _tc_mm1(x, w1, block_n=2000):
    n, f_in = x.shape
    hid = w1.shape[1]
    grid = n // block_n

    def body(x_ref, w_ref, out_ref):
        out_ref[...] = jnp.dot(x_ref[...], w_ref[...],
                               preferred_element_type=jnp.float32)

    return pl.pallas_call(
        body,
        grid=(grid,),
        in_specs=[
            pl.BlockSpec((block_n, f_in), lambda i: (i, 0)),
            pl.BlockSpec((f_in, hid), lambda i: (0, 0)),
        ],
        out_specs=pl.BlockSpec((block_n, hid), lambda i: (i, 0)),
        out_shape=jax.ShapeDtypeStruct((N_PAD, hid), jnp.float32),
    )(x, w1)


def _tc_final(q0, q1, hs2, dinv, w2, b2, block_n=2000):
    n, hid = hs2.shape
    c_out = w2.shape[1]
    grid = n // block_n

    def body(q0_ref, q1_ref, hs_ref, dinv_ref, w2_ref, b2_ref, out_ref):
        t = dinv_ref[...] * (q0_ref[...] + q1_ref[...] + hs_ref[...])
        logits = jnp.dot(t, w2_ref[...],
                         preferred_element_type=jnp.float32) + b2_ref[...]
        m = jnp.max(logits, axis=1, keepdims=True)
        lse = jnp.log(jnp.sum(jnp.exp(logits - m), axis=1, keepdims=True)) + m
        out_ref[...] = logits - lse

    return pl.pallas_call(
        body,
        grid=(grid,),
        in_specs=[
            pl.BlockSpec((block_n, hid), lambda i: (i, 0)),
            pl.BlockSpec((block_n, hid), lambda i: (i, 0)),
            pl.BlockSpec((block_n, hid), lambda i: (i, 0)),
            pl.BlockSpec((block_n, 1), lambda i: (i, 0)),
            pl.BlockSpec((hid, c_out), lambda i: (0, 0)),
            pl.BlockSpec((1, c_out), lambda i: (0, 0)),
        ],
        out_specs=pl.BlockSpec((block_n, c_out), lambda i: (i, 0)),
        out_shape=jax.ShapeDtypeStruct((n, c_out), jnp.float32),
    )(q0, q1, hs2, dinv, w2, b2)


def kernel(x, edge_index, W1, b1, W2, b2):
    n, f_in = x.shape
    e = edge_index.shape[1]
    hid = W1.shape[1]

    e_pad = ((e + CHUNK * NW - 1) // (CHUNK * NW)) * (CHUNK * NW)
    pad = e_pad - e
    src_full = jnp.concatenate([edge_index[0], jnp.zeros((pad,), jnp.int32)])
    dst_full = jnp.concatenate(
        [edge_index[1], jnp.full((pad,), n, jnp.int32)])
    src2d = src_full.reshape(e_pad // CHUNK, CHUNK)
    dst2d = dst_full.reshape(e_pad // CHUNK, CHUNK)

    z16 = jnp.zeros((N_PAD, hid), jnp.float32)
    zdeg = jnp.zeros((N_PAD_DEG,), jnp.float32)

    hw1 = _tc_mm1(x, W1)
    p, hs1x2, dinv_flat = _sc_mega1(hw1, src2d, dst2d, z16, zdeg)
    del hs1x2
    q, hs2x2 = _sc_mega2(p, hw1, dinv_flat, b1, src2d, dst2d, z16)
    dinv = dinv_flat[:n].reshape(n, 1)
    return _tc_final(q[0, :n], q[1, :n], hs2x2[0, :n], dinv, W2,
                     b2.reshape(1, W2.shape[1]))
